# baseline (speedup 1.0000x reference)
import functools
import math

import jax
import jax.numpy as jnp
from jax import lax
from jax.experimental import pallas as pl
from jax.experimental.pallas import tpu as pltpu
from jax.experimental.pallas import tpu_sc as plsc

H, W = 640, 480
N = 1048576
V = H * W

NC, NS, L = 2, 16, 16
NW = NC * NS
PPW = N // NW
C = 2048
G = C // L
NCHUNK = PPW // C

_INV_2PI = 1.0 / (2.0 * math.pi)
_SIN_C = (6.283185031955601, -41.34161602728077, 81.60091368067941,
          -76.62655311504956, 41.40344460088556, -12.57638987827264)


def _sin2pi(z):
    zc = jnp.clip(z, -16777216.0, 16777216.0)
    n = zc.astype(jnp.int32).astype(jnp.float32)
    f = zc - n
    f = f - jnp.where(f > 0.5, 1.0, 0.0)
    f = f + jnp.where(f < -0.5, 1.0, 0.0)
    f2 = f * f
    p = jnp.float32(_SIN_C[5])
    for c in (_SIN_C[4], _SIN_C[3], _SIN_C[2], _SIN_C[1], _SIN_C[0]):
        p = p * f2 + c
    return f * p


def _hi(v):
    return lax.bitcast_convert_type(v & jnp.int32(-65536), jnp.float32)


def _lo(v):
    return lax.bitcast_convert_type(v << 16, jnp.float32)


def _sc_body(t_hbm, x_hbm, y_hbm, u_hbm, uxx_hbm, uyy_hbm,
             tA_hbm, tB_hbm, tC_hbm, tD_hbm,
             out_hbm,
             x2, y2, t2, u2, uxx2, uyy2, idx2,
             pA2, pB2, pC2, pD2, out2,
             sA_v, sB_v, sC_v, sD_v,
             sxy0, sxy1, stu0, stu1, sg0, sg1, so0, so1):
    wid = lax.axis_index("s") * NC + lax.axis_index("c")
    sid = lax.axis_index("s")
    tabs = (sA_v, sB_v, sC_v, sD_v)
    pvs = (pA2, pB2, pC2, pD2)
    sxy, stu, sg, so = (sxy0, sxy1), (stu0, stu1), (sg0, sg1), (so0, so1)

    @pl.when(sid == 0)
    def _stage():
        pltpu.sync_copy(tA_hbm, sA_v)
        pltpu.sync_copy(tB_hbm, sB_v)
        pltpu.sync_copy(tC_hbm, sC_v)
        pltpu.sync_copy(tD_hbm, sD_v)

    plsc.subcore_barrier()

    def hbm_slice(i):
        return pl.ds(wid * PPW + i * C, C)

    def fire_xy(i):
        b, sl = i % 2, hbm_slice(i)
        return [pltpu.async_copy(x_hbm.at[sl], x2.at[b], sxy[b]),
                pltpu.async_copy(y_hbm.at[sl], y2.at[b], sxy[b])]

    def fire_tu(i):
        b, sl = i % 2, hbm_slice(i)
        return [pltpu.async_copy(t_hbm.at[sl], t2.at[b], stu[b]),
                pltpu.async_copy(u_hbm.at[sl], u2.at[b], stu[b]),
                pltpu.async_copy(uxx_hbm.at[sl], uxx2.at[b], stu[b]),
                pltpu.async_copy(uyy_hbm.at[sl], uyy2.at[b], stu[b])]

    def do_idx(i):
        b = i % 2

        def g1(g, carry1):
            s = pl.ds(g * L, L)
            xi = x2[b, s].astype(jnp.int32)
            yi = y2[b, s].astype(jnp.int32)
            xi = jnp.where(xi < 0, xi + H, xi)
            yi = jnp.where(yi < 0, yi + W, yi)
            xi = jnp.clip(xi, 0, H - 1)
            yi = jnp.clip(yi, 0, W - 1)
            fi = xi * W + yi
            idx2[b, s] = (fi & 7) * (V // 8) + (fi >> 3)
            return carry1

        lax.fori_loop(0, G, g1, 0)

    def fire_gather(i):
        b = i % 2
        return [pltpu.async_copy(tab.at[idx2.at[b]], pv.at[b], sg[b])
                for tab, pv in zip(tabs, pvs)]

    def physics(i):
        b = i % 2

        def g2(g, carry2):
            s = pl.ds(g * L, L)
            t = t2[b, s]
            u = u2[b, s]
            uxx = uxx2[b, s]
            uyy = uyy2[b, s]
            vA, vB = pA2[b, s], pB2[b, s]
            vC, vD = pC2[b, s], pD2[b, s]
            a1r = jnp.maximum(_hi(vA), 0.0)
            a4r = jnp.maximum(_lo(vA), 0.0)
            a5r = jnp.maximum(_hi(vB), 0.0)
            a9r = jnp.maximum(_lo(vB), 0.0)
            acc = 0.12 * a5r * (uxx + uyy)
            vessel = (uxx + uxx) < -0.5
            acc = acc + jnp.where(vessel, a1r * (37.0 - u), 0.0)
            acc = acc + 0.003 * a4r * jnp.exp((u - 37.0) * 0.1)
            acc = acc + _hi(vC) * _sin2pi(0.1 * t + _lo(vC) * _INV_2PI)
            acc = acc + _hi(vD) * _sin2pi(0.25 * t + _lo(vD) * _INV_2PI)
            acc = acc + a9r * (21.0 - u)
            out2[b, s] = acc
            return carry2

        lax.fori_loop(0, G, g2, 0)

    def fire_out(i):
        b = i % 2
        return [pltpu.async_copy(out2.at[b], out_hbm.at[hbm_slice(i)], so[b])]

    hxy, htu, hg, ho = {}, {}, {}, {}
    hxy[0] = fire_xy(0)
    for h in hxy[0]:
        h.wait()
    do_idx(0)
    hg[0] = fire_gather(0)
    hxy[1] = fire_xy(1)
    htu[0] = fire_tu(0)
    for i in range(NCHUNK):
        if i + 1 < NCHUNK:
            for h in hxy[i + 1]:
                h.wait()
            do_idx(i + 1)
            hg[i + 1] = fire_gather(i + 1)
            if i + 2 < NCHUNK:
                hxy[i + 2] = fire_xy(i + 2)
            htu[i + 1] = fire_tu(i + 1)
        for h in hg[i]:
            h.wait()
        for h in htu[i]:
            h.wait()
        if i >= 2:
            for h in ho[i - 2]:
                h.wait()
        physics(i)
        ho[i] = fire_out(i)
    for i in (NCHUNK - 2, NCHUNK - 1):
        for h in ho[i]:
            h.wait()


_sc_kernel = functools.partial(
    pl.kernel,
    mesh=plsc.VectorSubcoreMesh(core_axis_name="c", subcore_axis_name="s"),
    compiler_params=pltpu.CompilerParams(use_tc_tiling_on_sc=False),
    out_type=jax.ShapeDtypeStruct((N,), jnp.float32),
    scratch_types=(
        [pltpu.VMEM((2, C), jnp.float32)] * 6
        + [pltpu.VMEM((2, C), jnp.int32)]
        + [pltpu.VMEM((2, C), jnp.int32)] * 4
        + [pltpu.VMEM((2, C), jnp.float32)]
        + [pltpu.VMEM_SHARED((V,), jnp.int32)] * 4
        + [pltpu.SemaphoreType.DMA] * 8
    ),
)(_sc_body)


def _pack2(a, b):
    ha = lax.bitcast_convert_type(a.reshape(V).astype(jnp.bfloat16),
                                  jnp.uint16).astype(jnp.uint32)
    hb = lax.bitcast_convert_type(b.reshape(V).astype(jnp.bfloat16),
                                  jnp.uint16).astype(jnp.uint32)
    packed = lax.bitcast_convert_type((ha << 16) | hb, jnp.int32)
    return packed.reshape(V // 8, 8).T.reshape(V)


@jax.jit
def kernel(derivatives, a_1, a_2, a_3, a_4, a_5, a_6, a_7, a_9):
    cols = derivatives[:, 2:8].T
    return _sc_kernel(
        cols[0], cols[1], cols[2], cols[3], cols[4], cols[5],
        _pack2(a_1, a_4), _pack2(a_5, a_9),
        _pack2(a_2, a_3), _pack2(a_6, a_7))


# speedup vs baseline: 1.3025x; 1.3025x over previous
import functools
import math

import jax
import jax.numpy as jnp
from jax import lax
from jax.experimental import pallas as pl
from jax.experimental.pallas import tpu as pltpu
from jax.experimental.pallas import tpu_sc as plsc

H, W = 640, 480
N = 1048576
V = H * W

NC, NS, L = 2, 16, 16
NW = NC * NS
PPW = N // NW
C = 2048
G = C // L
NCHUNK = PPW // C

_INV_2PI = 1.0 / (2.0 * math.pi)
_SIN_C = (6.283185031955601, -41.34161602728077, 81.60091368067941,
          -76.62655311504956, 41.40344460088556, -12.57638987827264)


def _sin2pi(z):
    zc = jnp.clip(z, -16777216.0, 16777216.0)
    n = zc.astype(jnp.int32).astype(jnp.float32)
    f = zc - n
    f = f - jnp.where(f > 0.5, 1.0, 0.0)
    f = f + jnp.where(f < -0.5, 1.0, 0.0)
    f2 = f * f
    p = jnp.float32(_SIN_C[5])
    for c in (_SIN_C[4], _SIN_C[3], _SIN_C[2], _SIN_C[1], _SIN_C[0]):
        p = p * f2 + c
    return f * p


def _hi(v):
    return lax.bitcast_convert_type(v & jnp.int32(-65536), jnp.float32)


def _lo(v):
    return lax.bitcast_convert_type(v << 16, jnp.float32)


def _sc_body(t_hbm, x_hbm, y_hbm, u_hbm, uxx_hbm, uyy_hbm,
             tA_hbm, tB_hbm, tC_hbm, tD_hbm,
             out_hbm,
             x2, y2, t2, u2, uxx2, uyy2, idx2,
             pA2, pB2, pC2, pD2, out2,
             sA_v, sB_v, sC_v, sD_v,
             sxy0, sxy1, stu0, stu1, sg0, sg1, so0, so1):
    wid = lax.axis_index("s") * NC + lax.axis_index("c")
    sid = lax.axis_index("s")
    tabs = (sA_v, sB_v, sC_v, sD_v)
    pvs = (pA2, pB2, pC2, pD2)
    sxy, stu, sg, so = (sxy0, sxy1), (stu0, stu1), (sg0, sg1), (so0, so1)

    @pl.when(sid == 0)
    def _stage():
        pltpu.sync_copy(tA_hbm, sA_v)
        pltpu.sync_copy(tB_hbm, sB_v)
        pltpu.sync_copy(tC_hbm, sC_v)
        pltpu.sync_copy(tD_hbm, sD_v)

    plsc.subcore_barrier()

    def hbm_slice(i):
        return pl.ds(wid * PPW + i * C, C)

    def fire_xy(i):
        b, sl = i % 2, hbm_slice(i)
        return [pltpu.async_copy(x_hbm.at[sl], x2.at[b], sxy[b]),
                pltpu.async_copy(y_hbm.at[sl], y2.at[b], sxy[b])]

    def fire_tu(i):
        b, sl = i % 2, hbm_slice(i)
        return [pltpu.async_copy(t_hbm.at[sl], t2.at[b], stu[b]),
                pltpu.async_copy(u_hbm.at[sl], u2.at[b], stu[b]),
                pltpu.async_copy(uxx_hbm.at[sl], uxx2.at[b], stu[b]),
                pltpu.async_copy(uyy_hbm.at[sl], uyy2.at[b], stu[b])]

    def do_idx(i):
        b = i % 2

        def g1(g, carry1):
            s = pl.ds(g * L, L)
            xi = x2[b, s].astype(jnp.int32)
            yi = y2[b, s].astype(jnp.int32)
            xi = jnp.where(xi < 0, xi + H, xi)
            yi = jnp.where(yi < 0, yi + W, yi)
            xi = jnp.clip(xi, 0, H - 1)
            yi = jnp.clip(yi, 0, W - 1)
            idx2[b, s] = xi * W + yi
            return carry1

        lax.fori_loop(0, G, g1, 0)

    def fire_gather(i):
        b = i % 2
        return [pltpu.async_copy(tab.at[idx2.at[b]], pv.at[b], sg[b])
                for tab, pv in zip(tabs, pvs)]

    def physics(i):
        b = i % 2

        def g2(g, carry2):
            s = pl.ds(g * L, L)
            t = t2[b, s]
            u = u2[b, s]
            uxx = uxx2[b, s]
            uyy = uyy2[b, s]
            vA, vB = pA2[b, s], pB2[b, s]
            vC, vD = pC2[b, s], pD2[b, s]
            a1r = jnp.maximum(_hi(vA), 0.0)
            a4r = jnp.maximum(_lo(vA), 0.0)
            a5r = jnp.maximum(_hi(vB), 0.0)
            a9r = jnp.maximum(_lo(vB), 0.0)
            acc = 0.12 * a5r * (uxx + uyy)
            vessel = (uxx + uxx) < -0.5
            acc = acc + jnp.where(vessel, a1r * (37.0 - u), 0.0)
            acc = acc + 0.003 * a4r * jnp.exp((u - 37.0) * 0.1)
            acc = acc + _hi(vC) * _sin2pi(0.1 * t + _lo(vC) * _INV_2PI)
            acc = acc + _hi(vD) * _sin2pi(0.25 * t + _lo(vD) * _INV_2PI)
            acc = acc + a9r * (21.0 - u)
            out2[b, s] = acc
            return carry2

        lax.fori_loop(0, G, g2, 0)

    def fire_out(i):
        b = i % 2
        return [pltpu.async_copy(out2.at[b], out_hbm.at[hbm_slice(i)], so[b])]

    hxy, htu, hg, ho = {}, {}, {}, {}
    hxy[0] = fire_xy(0)
    for h in hxy[0]:
        h.wait()
    do_idx(0)
    hg[0] = fire_gather(0)
    hxy[1] = fire_xy(1)
    htu[0] = fire_tu(0)
    for i in range(NCHUNK):
        if i + 1 < NCHUNK:
            for h in hxy[i + 1]:
                h.wait()
            do_idx(i + 1)
            hg[i + 1] = fire_gather(i + 1)
            if i + 2 < NCHUNK:
                hxy[i + 2] = fire_xy(i + 2)
            htu[i + 1] = fire_tu(i + 1)
        for h in hg[i]:
            h.wait()
        for h in htu[i]:
            h.wait()
        if i >= 2:
            for h in ho[i - 2]:
                h.wait()
        physics(i)
        ho[i] = fire_out(i)
    for i in (NCHUNK - 2, NCHUNK - 1):
        for h in ho[i]:
            h.wait()


_sc_kernel = functools.partial(
    pl.kernel,
    mesh=plsc.VectorSubcoreMesh(core_axis_name="c", subcore_axis_name="s"),
    compiler_params=pltpu.CompilerParams(use_tc_tiling_on_sc=False),
    out_type=jax.ShapeDtypeStruct((N,), jnp.float32),
    scratch_types=(
        [pltpu.VMEM((2, C), jnp.float32)] * 6
        + [pltpu.VMEM((2, C), jnp.int32)]
        + [pltpu.VMEM((2, C), jnp.int32)] * 4
        + [pltpu.VMEM((2, C), jnp.float32)]
        + [pltpu.VMEM_SHARED((V,), jnp.int32)] * 4
        + [pltpu.SemaphoreType.DMA] * 8
    ),
)(_sc_body)


def _pack2(a, b):
    ha = lax.bitcast_convert_type(a.reshape(V).astype(jnp.bfloat16),
                                  jnp.uint16).astype(jnp.uint32)
    hb = lax.bitcast_convert_type(b.reshape(V).astype(jnp.bfloat16),
                                  jnp.uint16).astype(jnp.uint32)
    return lax.bitcast_convert_type((ha << 16) | hb, jnp.int32)


@jax.jit
def kernel(derivatives, a_1, a_2, a_3, a_4, a_5, a_6, a_7, a_9):
    cols = derivatives[:, 2:8].T
    return _sc_kernel(
        cols[0], cols[1], cols[2], cols[3], cols[4], cols[5],
        _pack2(a_1, a_4), _pack2(a_5, a_9),
        _pack2(a_2, a_3), _pack2(a_6, a_7))
